# baseline (speedup 1.0000x reference)
import functools

import jax
import jax.numpy as jnp
from jax import lax
from jax.experimental import pallas as pl
from jax.experimental.pallas import tpu as pltpu
from jax.experimental.pallas import tpu_sc as plsc

N = 10000
E = 320000
D = 128
G = 64

NC = 2
NS = 16
NW = NC * NS
NP = 10240
TILE_E = 10240
EP = NW * TILE_E
CH = 128
NCHUNK = TILE_E // CH
CHA = 64
NCHA = TILE_E // CHA
DEPTH = 4
ROWS_PER_TILE = NP // NS
WCH = ROWS_PER_TILE // CHA

R = 1024
NBLK = NP // R

def _zero_rows(ref, nrows):
    def body(i, _):
        for j in range(D // 16):
            ref[i, pl.ds(j * 16, 16)] = jnp.zeros((16,), jnp.float32)
        return 0
    lax.fori_loop(0, nrows, body, 0)


def _sc_deg_body(dst_hbm, d0_hbm, d1_hbm, dst_all, ones_v, stage_v, acc_sh):
    c = lax.axis_index("c")
    s = lax.axis_index("s")
    w = s * NC + c

    def init16(i, _):
        ones_v[pl.ds(i * 16, 16)] = jnp.ones((16,), jnp.float32)
        return 0
    lax.fori_loop(0, CH // 16, init16, 0)

    def zero16(i, _):
        stage_v[pl.ds(i * 16, 16)] = jnp.zeros((16,), jnp.float32)
        return 0
    lax.fori_loop(0, ROWS_PER_TILE // 16, zero16, 0)
    pltpu.sync_copy(stage_v, acc_sh.at[pl.ds(s * ROWS_PER_TILE, ROWS_PER_TILE)])
    pltpu.sync_copy(dst_hbm.at[w], dst_all)
    plsc.subcore_barrier()

    def chunk(i, _):
        pltpu.sync_copy(ones_v, acc_sh.at[dst_all.at[i]], add=True)
        return 0
    lax.fori_loop(0, NCHUNK, chunk, 0)
    plsc.subcore_barrier()

    r0 = s * ROWS_PER_TILE
    pltpu.sync_copy(acc_sh.at[pl.ds(r0, ROWS_PER_TILE)], stage_v)

    @pl.when(c == 0)
    def _():
        pltpu.sync_copy(stage_v, d0_hbm.at[pl.ds(r0, ROWS_PER_TILE)])

    @pl.when(c == 1)
    def _():
        pltpu.sync_copy(stage_v, d1_hbm.at[pl.ds(r0, ROWS_PER_TILE)])


def _sc_agg_body(y_hbm, src_hbm, dst_hbm, p0_hbm, p1_hbm,
                 s0, s1, s2, s3, d0, d1, d2, d3, b0, b1, b2, b3, acc_sh,
                 g0, g1, g2, g3, t0, t1, t2, t3):
    c = lax.axis_index("c")
    s = lax.axis_index("s")
    w = s * NC + c
    sbufs = (s0, s1, s2, s3)
    dbufs = (d0, d1, d2, d3)
    bufs = (b0, b1, b2, b3)
    gsems = (g0, g1, g2, g3)
    ssems = (t0, t1, t2, t3)

    _zero_rows(b0, CHA)
    for k in range(WCH):
        pltpu.sync_copy(b0, acc_sh.at[pl.ds(s * ROWS_PER_TILE + k * CHA, CHA)])
    plsc.subcore_barrier()

    base = w * TILE_E

    def gather(i, b):
        off = base + i * CHA
        pltpu.sync_copy(src_hbm.at[pl.ds(off, CHA)], sbufs[b])
        pltpu.sync_copy(dst_hbm.at[pl.ds(off, CHA)], dbufs[b].at[0])
        pltpu.async_copy(y_hbm.at[sbufs[b]], bufs[b], gsems[b])

    def gwait(b):
        pltpu.make_async_copy(y_hbm.at[pl.ds(0, CHA)], bufs[b], gsems[b]).wait()

    def scat(i, b):
        pltpu.async_copy(bufs[b], acc_sh.at[dbufs[b].at[0]], ssems[b],
                         add=True)

    def swait(b):
        pltpu.make_async_copy(bufs[b], acc_sh.at[pl.ds(0, CHA)],
                              ssems[b]).wait()

    for b in range(DEPTH):
        gather(b, b)

    def group(j, _):
        i0 = DEPTH * j
        for b in range(DEPTH):
            gwait(b)
            scat(i0 + b, b)
        for b in range(DEPTH):
            swait(b)
            gather(i0 + DEPTH + b, b)
        return 0
    lax.fori_loop(0, NCHA // DEPTH - 1, group, 0)

    i0 = NCHA - DEPTH
    for b in range(DEPTH):
        gwait(b)
        scat(i0 + b, b)
    for b in range(DEPTH):
        swait(b)
    plsc.subcore_barrier()

    for k in range(WCH):
        r0 = s * ROWS_PER_TILE + k * CHA
        pltpu.sync_copy(acc_sh.at[pl.ds(r0, CHA)], b0)

        @pl.when(c == 0)
        def _():
            pltpu.sync_copy(b0, p0_hbm.at[pl.ds(r0, CHA)])

        @pl.when(c == 1)
        def _():
            pltpu.sync_copy(b0, p1_hbm.at[pl.ds(r0, CHA)])


@functools.cache
def _sc_kernels():
    mesh = plsc.VectorSubcoreMesh(core_axis_name="c", subcore_axis_name="s",
                                  num_cores=NC, num_subcores=NS)
    sc_deg = pl.kernel(
        _sc_deg_body,
        out_type=(jax.ShapeDtypeStruct((NP,), jnp.float32),
                  jax.ShapeDtypeStruct((NP,), jnp.float32)),
        mesh=mesh,
        scratch_types=[
            pltpu.VMEM((NCHUNK, CH), jnp.int32),
            pltpu.VMEM((CH,), jnp.float32),
            pltpu.VMEM((ROWS_PER_TILE,), jnp.float32),
            pltpu.VMEM_SHARED((NP,), jnp.float32),
        ],
    )
    sc_agg = pl.kernel(
        _sc_agg_body,
        out_type=(jax.ShapeDtypeStruct((NP, D), jnp.float32),
                  jax.ShapeDtypeStruct((NP, D), jnp.float32)),
        mesh=mesh,
        scratch_types=(
            [pltpu.VMEM((CHA,), jnp.int32) for _ in range(DEPTH)]
            + [pltpu.VMEM((1, CHA), jnp.int32) for _ in range(DEPTH)]
            + [pltpu.VMEM((CHA, D), jnp.float32) for _ in range(DEPTH)]
            + [pltpu.VMEM_SHARED((NP, D), jnp.float32)]
            + [pltpu.SemaphoreType.DMA for _ in range(2 * DEPTH)]
        ),
    )
    return sc_deg, sc_agg


def _t0_body(d0_ref, d1_ref, x_ref, w_ref, dinv_ref, y1_ref):
    deg = d0_ref[...] + d1_ref[...] + 1.0
    dv = lax.rsqrt(deg)
    dinv_ref[...] = dv
    y1_ref[...] = dv * jnp.dot(x_ref[...], w_ref[...],
                               preferred_element_type=jnp.float32)


def _tc_t0(d0, d1, x, W1):
    return pl.pallas_call(
        _t0_body,
        grid=(NBLK,),
        in_specs=[
            pl.BlockSpec((R, 1), lambda i: (i, 0)),
            pl.BlockSpec((R, 1), lambda i: (i, 0)),
            pl.BlockSpec((R, D), lambda i: (i, 0)),
            pl.BlockSpec((D, D), lambda i: (0, 0)),
        ],
        out_specs=[
            pl.BlockSpec((R, 1), lambda i: (i, 0)),
            pl.BlockSpec((R, D), lambda i: (i, 0)),
        ],
        out_shape=[
            jax.ShapeDtypeStruct((NP, 1), jnp.float32),
            jax.ShapeDtypeStruct((NP, D), jnp.float32),
        ],
    )(d0, d1, x, W1)


def _tl_body(p0_ref, p1_ref, yk_ref, dinv_ref, res_ref, b_ref, w_ref,
             h_ref, ynext_ref):
    dv = dinv_ref[...]
    h = dv * (p0_ref[...] + p1_ref[...] + yk_ref[...]) + b_ref[...]
    h_ref[...] = h
    r = jnp.maximum(h + res_ref[...], 0.0)
    ynext_ref[...] = dv * jnp.dot(r, w_ref[...],
                                  preferred_element_type=jnp.float32)


def _tc_layer(p0, p1, yk, dinv, res, b, Wn):
    return pl.pallas_call(
        _tl_body,
        grid=(NBLK,),
        in_specs=[
            pl.BlockSpec((R, D), lambda i: (i, 0)),
            pl.BlockSpec((R, D), lambda i: (i, 0)),
            pl.BlockSpec((R, D), lambda i: (i, 0)),
            pl.BlockSpec((R, 1), lambda i: (i, 0)),
            pl.BlockSpec((R, D), lambda i: (i, 0)),
            pl.BlockSpec((1, D), lambda i: (0, 0)),
            pl.BlockSpec((D, D), lambda i: (0, 0)),
        ],
        out_specs=[
            pl.BlockSpec((R, D), lambda i: (i, 0)),
            pl.BlockSpec((R, D), lambda i: (i, 0)),
        ],
        out_shape=[
            jax.ShapeDtypeStruct((NP, D), jnp.float32),
            jax.ShapeDtypeStruct((NP, D), jnp.float32),
        ],
    )(p0, p1, yk, dinv, res, b, Wn)


def _t3_body(p0_ref, p1_ref, y3_ref, dinv_ref, res_ref, b_ref, batch_ref,
             out_ref, seg_acc, cnt_acc):
    i = pl.program_id(0)

    @pl.when(i == 0)
    def _():
        seg_acc[...] = jnp.zeros_like(seg_acc)
        cnt_acc[...] = jnp.zeros_like(cnt_acc)

    hf = (dinv_ref[...] * (p0_ref[...] + p1_ref[...] + y3_ref[...])
          + b_ref[...] + res_ref[...])
    bb = batch_ref[...].reshape(1, R)
    onehot = (lax.broadcasted_iota(jnp.int32, (G, R), 0) == bb
              ).astype(jnp.float32)
    seg_acc[...] += jnp.dot(onehot, hf, preferred_element_type=jnp.float32)
    cnt_acc[...] += jnp.sum(onehot, axis=1, keepdims=True)

    @pl.when(i == pl.num_programs(0) - 1)
    def _():
        out_ref[...] = seg_acc[...] / jnp.maximum(cnt_acc[...], 1.0)


def _tc_final(p0, p1, y3, dinv, res, b, batch2d):
    return pl.pallas_call(
        _t3_body,
        grid=(NBLK,),
        in_specs=[
            pl.BlockSpec((R, D), lambda i: (i, 0)),
            pl.BlockSpec((R, D), lambda i: (i, 0)),
            pl.BlockSpec((R, D), lambda i: (i, 0)),
            pl.BlockSpec((R, 1), lambda i: (i, 0)),
            pl.BlockSpec((R, D), lambda i: (i, 0)),
            pl.BlockSpec((1, D), lambda i: (0, 0)),
            pl.BlockSpec((R, 1), lambda i: (i, 0)),
        ],
        out_specs=pl.BlockSpec((G, D), lambda i: (0, 0)),
        out_shape=jax.ShapeDtypeStruct((G, D), jnp.float32),
        scratch_shapes=[
            pltpu.VMEM((G, D), jnp.float32),
            pltpu.VMEM((G, 1), jnp.float32),
        ],
    )(p0, p1, y3, dinv, res, b, batch2d)


def kernel(x, edge_index, batch, W1, b1, W2, b2, W3, b3):
    src = edge_index[0]
    dst = edge_index[1]
    pad_e = EP - E
    pad_ids = jnp.arange(pad_e, dtype=jnp.int32)
    src_p = jnp.concatenate([src, pad_ids % N])
    dst_p = jnp.concatenate([dst, N + pad_ids % (NP - N)])
    dst_deg = dst_p.reshape(NW, NCHUNK, CH)
    x_p = jnp.pad(x, ((0, NP - N), (0, 0)))
    batch2d = jnp.pad(batch, (0, NP - N), constant_values=G).reshape(NP, 1)
    b1r = b1.reshape(1, D)
    b2r = b2.reshape(1, D)
    b3r = b3.reshape(1, D)

    sc_deg, sc_agg = _sc_kernels()
    d0, d1 = sc_deg(dst_deg)
    dinv, y1 = _tc_t0(d0.reshape(NP, 1), d1.reshape(NP, 1), x_p, W1)

    p0, p1 = sc_agg(y1, src_p, dst_p)
    h1, y2 = _tc_layer(p0, p1, y1, dinv, x_p, b1r, W2)

    p0, p1 = sc_agg(y2, src_p, dst_p)
    h2, y3 = _tc_layer(p0, p1, y2, dinv, h1, b2r, W3)

    p0, p1 = sc_agg(y3, src_p, dst_p)
    out = _tc_final(p0, p1, y3, dinv, h2, b3r, batch2d)
    return out


# speedup vs baseline: 1.3375x; 1.3375x over previous
import functools

import jax
import jax.numpy as jnp
from jax import lax
from jax.experimental import pallas as pl
from jax.experimental.pallas import tpu as pltpu
from jax.experimental.pallas import tpu_sc as plsc

N = 10000
E = 320000
D = 128
G = 64

NC = 2
NS = 16
NW = NC * NS
NP = 10240
TILE_E = 10240
EP = NW * TILE_E
CH = 128
NCHUNK = TILE_E // CH
CHA = 128
NCHA = TILE_E // CHA
DEPTH = 2
ROWS_PER_TILE = NP // NS
WCH = ROWS_PER_TILE // CHA

R = 1024
NBLK = NP // R

def _zero_rows(ref, nrows):
    def body(i, _):
        for j in range(D // 16):
            ref[i, pl.ds(j * 16, 16)] = jnp.zeros((16,), jnp.float32)
        return 0
    lax.fori_loop(0, nrows, body, 0)


def _sc_deg_body(dst_hbm, d0_hbm, d1_hbm, dst_all, ones_v, stage_v, acc_sh):
    c = lax.axis_index("c")
    s = lax.axis_index("s")
    w = s * NC + c

    def init16(i, _):
        ones_v[pl.ds(i * 16, 16)] = jnp.ones((16,), jnp.float32)
        return 0
    lax.fori_loop(0, CH // 16, init16, 0)

    def zero16(i, _):
        stage_v[pl.ds(i * 16, 16)] = jnp.zeros((16,), jnp.float32)
        return 0
    lax.fori_loop(0, ROWS_PER_TILE // 16, zero16, 0)
    pltpu.sync_copy(stage_v, acc_sh.at[pl.ds(s * ROWS_PER_TILE, ROWS_PER_TILE)])
    pltpu.sync_copy(dst_hbm.at[w], dst_all)
    plsc.subcore_barrier()

    def chunk(i, _):
        pltpu.sync_copy(ones_v, acc_sh.at[dst_all.at[i]], add=True)
        return 0
    lax.fori_loop(0, NCHUNK, chunk, 0)
    plsc.subcore_barrier()

    r0 = s * ROWS_PER_TILE
    pltpu.sync_copy(acc_sh.at[pl.ds(r0, ROWS_PER_TILE)], stage_v)

    @pl.when(c == 0)
    def _():
        pltpu.sync_copy(stage_v, d0_hbm.at[pl.ds(r0, ROWS_PER_TILE)])

    @pl.when(c == 1)
    def _():
        pltpu.sync_copy(stage_v, d1_hbm.at[pl.ds(r0, ROWS_PER_TILE)])


def _sc_agg_body(y_hbm, src_hbm, dst_hbm, p0_hbm, p1_hbm,
                 s0, s1, dst_all, b0, b1, acc_sh, g0, g1, t0, t1):
    c = lax.axis_index("c")
    s = lax.axis_index("s")
    w = s * NC + c
    sbufs = (s0, s1)
    bufs = (b0, b1)
    gsems = (g0, g1)
    ssems = (t0, t1)

    _zero_rows(b0, CHA)
    for k in range(WCH):
        pltpu.sync_copy(b0, acc_sh.at[pl.ds(s * ROWS_PER_TILE + k * CHA, CHA)])
    pltpu.sync_copy(dst_hbm.at[w], dst_all)
    plsc.subcore_barrier()

    base = w * TILE_E

    def gather(i, b):
        pltpu.sync_copy(src_hbm.at[pl.ds(base + i * CHA, CHA)], sbufs[b])
        pltpu.async_copy(y_hbm.at[sbufs[b]], bufs[b], gsems[b])

    def gwait(b):
        pltpu.make_async_copy(y_hbm.at[pl.ds(0, CHA)], bufs[b], gsems[b]).wait()

    def scat(i, b):
        pltpu.async_copy(bufs[b], acc_sh.at[dst_all.at[i]], ssems[b],
                         add=True)

    def swait(b):
        pltpu.make_async_copy(bufs[b], acc_sh.at[pl.ds(0, CHA)],
                              ssems[b]).wait()

    gather(0, 0)
    gather(1, 1)

    def pair(j, _):
        i0 = 2 * j
        gwait(0)
        scat(i0, 0)
        gwait(1)
        scat(i0 + 1, 1)
        swait(0)
        gather(i0 + 2, 0)
        swait(1)
        gather(i0 + 3, 1)
        return 0
    lax.fori_loop(0, NCHA // 2 - 1, pair, 0)

    i0 = NCHA - 2
    gwait(0)
    scat(i0, 0)
    gwait(1)
    scat(i0 + 1, 1)
    swait(0)
    swait(1)
    plsc.subcore_barrier()

    for k in range(WCH):
        r0 = s * ROWS_PER_TILE + k * CHA
        pltpu.sync_copy(acc_sh.at[pl.ds(r0, CHA)], b0)

        @pl.when(c == 0)
        def _():
            pltpu.sync_copy(b0, p0_hbm.at[pl.ds(r0, CHA)])

        @pl.when(c == 1)
        def _():
            pltpu.sync_copy(b0, p1_hbm.at[pl.ds(r0, CHA)])


@functools.cache
def _sc_kernels():
    mesh = plsc.VectorSubcoreMesh(core_axis_name="c", subcore_axis_name="s",
                                  num_cores=NC, num_subcores=NS)
    sc_deg = pl.kernel(
        _sc_deg_body,
        out_type=(jax.ShapeDtypeStruct((NP,), jnp.float32),
                  jax.ShapeDtypeStruct((NP,), jnp.float32)),
        mesh=mesh,
        scratch_types=[
            pltpu.VMEM((NCHUNK, CH), jnp.int32),
            pltpu.VMEM((CH,), jnp.float32),
            pltpu.VMEM((ROWS_PER_TILE,), jnp.float32),
            pltpu.VMEM_SHARED((NP,), jnp.float32),
        ],
    )
    sc_agg = pl.kernel(
        _sc_agg_body,
        out_type=(jax.ShapeDtypeStruct((NP, D), jnp.float32),
                  jax.ShapeDtypeStruct((NP, D), jnp.float32)),
        mesh=mesh,
        scratch_types=(
            [pltpu.VMEM((CHA,), jnp.int32) for _ in range(DEPTH)]
            + [pltpu.VMEM((NCHA, CHA), jnp.int32)]
            + [pltpu.VMEM((CHA, D), jnp.float32) for _ in range(DEPTH)]
            + [pltpu.VMEM_SHARED((NP, D), jnp.float32)]
            + [pltpu.SemaphoreType.DMA for _ in range(2 * DEPTH)]
        ),
    )
    return sc_deg, sc_agg


def _t0_body(d0_ref, d1_ref, x_ref, w_ref, dinv_ref, y1_ref):
    deg = d0_ref[...] + d1_ref[...] + 1.0
    dv = lax.rsqrt(deg)
    dinv_ref[...] = dv
    y1_ref[...] = dv * jnp.dot(x_ref[...], w_ref[...],
                               preferred_element_type=jnp.float32)


def _tc_t0(d0, d1, x, W1):
    return pl.pallas_call(
        _t0_body,
        grid=(NBLK,),
        in_specs=[
            pl.BlockSpec((R, 1), lambda i: (i, 0)),
            pl.BlockSpec((R, 1), lambda i: (i, 0)),
            pl.BlockSpec((R, D), lambda i: (i, 0)),
            pl.BlockSpec((D, D), lambda i: (0, 0)),
        ],
        out_specs=[
            pl.BlockSpec((R, 1), lambda i: (i, 0)),
            pl.BlockSpec((R, D), lambda i: (i, 0)),
        ],
        out_shape=[
            jax.ShapeDtypeStruct((NP, 1), jnp.float32),
            jax.ShapeDtypeStruct((NP, D), jnp.float32),
        ],
    )(d0, d1, x, W1)


def _tl_body(p0_ref, p1_ref, yk_ref, dinv_ref, res_ref, b_ref, w_ref,
             h_ref, ynext_ref):
    dv = dinv_ref[...]
    h = dv * (p0_ref[...] + p1_ref[...] + yk_ref[...]) + b_ref[...]
    h_ref[...] = h
    r = jnp.maximum(h + res_ref[...], 0.0)
    ynext_ref[...] = dv * jnp.dot(r, w_ref[...],
                                  preferred_element_type=jnp.float32)


def _tc_layer(p0, p1, yk, dinv, res, b, Wn):
    return pl.pallas_call(
        _tl_body,
        grid=(NBLK,),
        in_specs=[
            pl.BlockSpec((R, D), lambda i: (i, 0)),
            pl.BlockSpec((R, D), lambda i: (i, 0)),
            pl.BlockSpec((R, D), lambda i: (i, 0)),
            pl.BlockSpec((R, 1), lambda i: (i, 0)),
            pl.BlockSpec((R, D), lambda i: (i, 0)),
            pl.BlockSpec((1, D), lambda i: (0, 0)),
            pl.BlockSpec((D, D), lambda i: (0, 0)),
        ],
        out_specs=[
            pl.BlockSpec((R, D), lambda i: (i, 0)),
            pl.BlockSpec((R, D), lambda i: (i, 0)),
        ],
        out_shape=[
            jax.ShapeDtypeStruct((NP, D), jnp.float32),
            jax.ShapeDtypeStruct((NP, D), jnp.float32),
        ],
    )(p0, p1, yk, dinv, res, b, Wn)


def _t3_body(p0_ref, p1_ref, y3_ref, dinv_ref, res_ref, b_ref, batch_ref,
             out_ref, seg_acc, cnt_acc):
    i = pl.program_id(0)

    @pl.when(i == 0)
    def _():
        seg_acc[...] = jnp.zeros_like(seg_acc)
        cnt_acc[...] = jnp.zeros_like(cnt_acc)

    hf = (dinv_ref[...] * (p0_ref[...] + p1_ref[...] + y3_ref[...])
          + b_ref[...] + res_ref[...])
    bb = batch_ref[...].reshape(1, R)
    onehot = (lax.broadcasted_iota(jnp.int32, (G, R), 0) == bb
              ).astype(jnp.float32)
    seg_acc[...] += jnp.dot(onehot, hf, preferred_element_type=jnp.float32)
    cnt_acc[...] += jnp.sum(onehot, axis=1, keepdims=True)

    @pl.when(i == pl.num_programs(0) - 1)
    def _():
        out_ref[...] = seg_acc[...] / jnp.maximum(cnt_acc[...], 1.0)


def _tc_final(p0, p1, y3, dinv, res, b, batch2d):
    return pl.pallas_call(
        _t3_body,
        grid=(NBLK,),
        in_specs=[
            pl.BlockSpec((R, D), lambda i: (i, 0)),
            pl.BlockSpec((R, D), lambda i: (i, 0)),
            pl.BlockSpec((R, D), lambda i: (i, 0)),
            pl.BlockSpec((R, 1), lambda i: (i, 0)),
            pl.BlockSpec((R, D), lambda i: (i, 0)),
            pl.BlockSpec((1, D), lambda i: (0, 0)),
            pl.BlockSpec((R, 1), lambda i: (i, 0)),
        ],
        out_specs=pl.BlockSpec((G, D), lambda i: (0, 0)),
        out_shape=jax.ShapeDtypeStruct((G, D), jnp.float32),
        scratch_shapes=[
            pltpu.VMEM((G, D), jnp.float32),
            pltpu.VMEM((G, 1), jnp.float32),
        ],
    )(p0, p1, y3, dinv, res, b, batch2d)


def kernel(x, edge_index, batch, W1, b1, W2, b2, W3, b3):
    src = edge_index[0]
    dst = edge_index[1]
    pad_e = EP - E
    pad_ids = jnp.arange(pad_e, dtype=jnp.int32)
    src_p = jnp.concatenate([src, pad_ids % N])
    dst_p = jnp.concatenate([dst, N + pad_ids % (NP - N)])
    dst_deg = dst_p.reshape(NW, NCHUNK, CH)
    x_p = jnp.pad(x, ((0, NP - N), (0, 0)))
    batch2d = jnp.pad(batch, (0, NP - N), constant_values=G).reshape(NP, 1)
    b1r = b1.reshape(1, D)
    b2r = b2.reshape(1, D)
    b3r = b3.reshape(1, D)

    sc_deg, sc_agg = _sc_kernels()
    d0, d1 = sc_deg(dst_deg)
    dinv, y1 = _tc_t0(d0.reshape(NP, 1), d1.reshape(NP, 1), x_p, W1)

    dst_agg = dst_p.reshape(NW, NCHA, CHA)
    p0, p1 = sc_agg(y1, src_p, dst_agg)
    h1, y2 = _tc_layer(p0, p1, y1, dinv, x_p, b1r, W2)

    p0, p1 = sc_agg(y2, src_p, dst_agg)
    h2, y3 = _tc_layer(p0, p1, y2, dinv, h1, b2r, W3)

    p0, p1 = sc_agg(y3, src_p, dst_agg)
    out = _tc_final(p0, p1, y3, dinv, h2, b3r, batch2d)
    return out
